# baseline (speedup 1.0000x reference)
import functools

import jax
import jax.numpy as jnp
from jax import lax
from jax.experimental import pallas as pl
from jax.experimental.pallas import tpu as pltpu
from jax.experimental.pallas import tpu_sc as plsc

_VOCAB = 1_000_000
_EMB = 64
_B = 4096
_L = 50
_SCALE = 8.0

_NUM_CORES = 2
_NUM_SUBCORES = 16
_LANES = 16
_NW = _NUM_CORES * _NUM_SUBCORES

_B_TOTAL = _B * _L
_B_PER_W = _B_TOTAL // _NW
_CHUNK = 128
_N_CHUNKS = _B_PER_W // _CHUNK
_NBUF = 2


_TBLK = 16384


def _make_tc_widen_kernel():
  def body(tt_ref, wide_ref):
    eye = jnp.eye(_EMB, dtype=jnp.float32)
    t = lax.dot_general(
        tt_ref[...], eye, (((0,), (0,)), ((), ())),
        preferred_element_type=jnp.float32)
    wide_ref[...] = jnp.concatenate([t, t], axis=1)

  return pl.pallas_call(
      body,
      grid=(pl.cdiv(_VOCAB, _TBLK),),
      in_specs=[pl.BlockSpec((_EMB, _TBLK), lambda i: (0, i))],
      out_specs=pl.BlockSpec((_TBLK, 2 * _EMB), lambda i: (i, 0)),
      out_shape=jax.ShapeDtypeStruct((_VOCAB, 2 * _EMB), jnp.float32),
  )


_tc_widen = _make_tc_widen_kernel()


def _make_sc_kernel():
  mesh = plsc.VectorSubcoreMesh(
      core_axis_name="c",
      subcore_axis_name="s",
      num_cores=_NUM_CORES,
      num_subcores=_NUM_SUBCORES,
  )

  @functools.partial(
      pl.kernel,
      out_type=jax.ShapeDtypeStruct((_B_TOTAL, _EMB), jnp.float32),
      mesh=mesh,
      compiler_params=pltpu.CompilerParams(use_tc_tiling_on_sc=False),
      scratch_types=[
          pltpu.VMEM((_N_CHUNKS, _CHUNK), jnp.int32),
          pltpu.VMEM((_NBUF, _CHUNK, _EMB), jnp.float32),
          pltpu.VMEM((_NBUF, _CHUNK, _EMB), jnp.float32),
          pltpu.SemaphoreType.DMA,
          pltpu.SemaphoreType.DMA,
          pltpu.SemaphoreType.DMA,
          pltpu.SemaphoreType.DMA,
      ],
  )
  def emb_kernel(idx_hbm, table_hbm, out_hbm, idx_v, gbuf, wbuf,
                 gsem0, gsem1, wsem0, wsem1):
    gsems = (gsem0, gsem1)
    wsems = (wsem0, wsem1)
    wid = lax.axis_index("s") * _NUM_CORES + lax.axis_index("c")
    base = wid * _CHUNK

    pltpu.sync_copy(idx_hbm.at[wid], idx_v)
    def dbl_row(g, carry):
      for col in range(_CHUNK // _LANES):
        sl = pl.ds(col * _LANES, _LANES)
        idx_v[g, sl] = idx_v[g, sl] * 2
      return carry

    lax.fori_loop(0, _N_CHUNKS, dbl_row, 0)

    def gather_start(g, b):
      pltpu.async_copy(table_hbm.at[idx_v.at[g]], gbuf.at[b], gsems[b])

    def gather_wait(b):
      pltpu.make_async_copy(table_hbm.at[idx_v.at[0]], gbuf.at[b],
                            gsems[b]).wait()

    def scale(b):
      def row(r, carry):
        for col in range(_EMB // _LANES):
          sl = pl.ds(col * _LANES, _LANES)
          wbuf[b, r, sl] = gbuf[b, r, sl] * _SCALE
        return carry
      lax.fori_loop(0, _CHUNK, row, 0)

    def write_start(g, b):
      off = g * _B + base
      pltpu.async_copy(wbuf.at[b], out_hbm.at[pl.ds(off, _CHUNK)], wsems[b])

    def write_wait(b):
      pltpu.make_async_copy(wbuf.at[b], out_hbm.at[pl.ds(0, _CHUNK)],
                            wsems[b]).wait()

    for b in range(_NBUF):
      gather_start(b, b)
    for i in range(_NBUF):
      b = i % _NBUF
      gather_wait(b)
      scale(b)
      gather_start(i + _NBUF, b)
      write_start(i, b)

    n_steady = (_N_CHUNKS - 2 * _NBUF) // _NBUF

    def steady(t, carry):
      i0 = _NBUF + t * _NBUF
      for b in range(_NBUF):
        i = i0 + b
        gather_wait(b)
        write_wait(b)
        scale(b)
        gather_start(i + _NBUF, b)
        write_start(i, b)
      return carry

    lax.fori_loop(0, n_steady, steady, 0)

    for i in range(_N_CHUNKS - _NBUF, _N_CHUNKS):
      b = i % _NBUF
      gather_wait(b)
      write_wait(b)
      scale(b)
      write_start(i, b)
    for b in range(_NBUF):
      write_wait(b)

  return emb_kernel


_emb_kernel = _make_sc_kernel()


@jax.jit
def kernel(tokens, table):
  wide = _tc_widen(table.T)
  flat = wide.reshape(2 * _VOCAB, _EMB)
  idx = tokens.reshape(_NW, _CHUNK, _N_CHUNKS).transpose(0, 2, 1)
  out = _emb_kernel(idx, flat)
  return out.reshape(_L, _B, _EMB).transpose(1, 0, 2)


# speedup vs baseline: 1.0016x; 1.0016x over previous
import functools

import jax
import jax.numpy as jnp
from jax import lax
from jax.experimental import pallas as pl
from jax.experimental.pallas import tpu as pltpu
from jax.experimental.pallas import tpu_sc as plsc

_VOCAB = 1_000_000
_EMB = 64
_B = 4096
_L = 50
_SCALE = 8.0

_NUM_CORES = 2
_NUM_SUBCORES = 16
_LANES = 16
_NW = _NUM_CORES * _NUM_SUBCORES

_B_TOTAL = _B * _L
_B_PER_W = _B_TOTAL // _NW
_CHUNK = 128
_N_CHUNKS = _B_PER_W // _CHUNK
_NBUF = 2


_TBLK = 16384


def _make_tc_widen_kernel():
  def body(tt_ref, wide_ref):
    t = tt_ref[...].T
    wide_ref[...] = jnp.concatenate([t, t], axis=1)

  return pl.pallas_call(
      body,
      grid=(pl.cdiv(_VOCAB, _TBLK),),
      in_specs=[pl.BlockSpec((_EMB, _TBLK), lambda i: (0, i))],
      out_specs=pl.BlockSpec((_TBLK, 2 * _EMB), lambda i: (i, 0)),
      out_shape=jax.ShapeDtypeStruct((_VOCAB, 2 * _EMB), jnp.float32),
  )


_tc_widen = _make_tc_widen_kernel()


def _make_sc_kernel():
  mesh = plsc.VectorSubcoreMesh(
      core_axis_name="c",
      subcore_axis_name="s",
      num_cores=_NUM_CORES,
      num_subcores=_NUM_SUBCORES,
  )

  @functools.partial(
      pl.kernel,
      out_type=jax.ShapeDtypeStruct((_B_TOTAL, _EMB), jnp.float32),
      mesh=mesh,
      compiler_params=pltpu.CompilerParams(use_tc_tiling_on_sc=False),
      scratch_types=[
          pltpu.VMEM((_N_CHUNKS, _CHUNK), jnp.int32),
          pltpu.VMEM((_NBUF, _CHUNK, _EMB), jnp.float32),
          pltpu.VMEM((_NBUF, _CHUNK, _EMB), jnp.float32),
          pltpu.SemaphoreType.DMA,
          pltpu.SemaphoreType.DMA,
          pltpu.SemaphoreType.DMA,
          pltpu.SemaphoreType.DMA,
      ],
  )
  def emb_kernel(idx_hbm, table_hbm, out_hbm, idx_v, gbuf, wbuf,
                 gsem0, gsem1, wsem0, wsem1):
    gsems = (gsem0, gsem1)
    wsems = (wsem0, wsem1)
    wid = lax.axis_index("s") * _NUM_CORES + lax.axis_index("c")
    base = wid * _CHUNK

    pltpu.sync_copy(idx_hbm.at[wid], idx_v)
    def dbl_row(g, carry):
      for col in range(_CHUNK // _LANES):
        sl = pl.ds(col * _LANES, _LANES)
        idx_v[g, sl] = idx_v[g, sl] * 2
      return carry

    lax.fori_loop(0, _N_CHUNKS, dbl_row, 0)

    def gather_start(g, b):
      pltpu.async_copy(table_hbm.at[idx_v.at[g]], gbuf.at[b], gsems[b])

    def gather_wait(b):
      pltpu.make_async_copy(table_hbm.at[idx_v.at[0]], gbuf.at[b],
                            gsems[b]).wait()

    def scale(b):
      def row(r, carry):
        for col in range(_EMB // _LANES):
          sl = pl.ds(col * _LANES, _LANES)
          wbuf[b, r, sl] = gbuf[b, r, sl] * _SCALE
        return carry
      lax.fori_loop(0, _CHUNK, row, 0)

    def write_start(g, b):
      off = g * _B + base
      pltpu.async_copy(wbuf.at[b], out_hbm.at[pl.ds(off, _CHUNK)], wsems[b])

    def write_wait(b):
      pltpu.make_async_copy(wbuf.at[b], out_hbm.at[pl.ds(0, _CHUNK)],
                            wsems[b]).wait()

    for b in range(_NBUF):
      gather_start(b, b)
    for i in range(_NBUF):
      b = i % _NBUF
      gather_wait(b)
      scale(b)
      gather_start(i + _NBUF, b)
      write_start(i, b)

    n_steady = (_N_CHUNKS - 2 * _NBUF) // _NBUF

    def steady(t, carry):
      i0 = _NBUF + t * _NBUF
      for b in range(_NBUF):
        i = i0 + b
        gather_wait(b)
        write_wait(b)
        scale(b)
        gather_start(i + _NBUF, b)
        write_start(i, b)
      return carry

    lax.fori_loop(0, n_steady, steady, 0)

    for i in range(_N_CHUNKS - _NBUF, _N_CHUNKS):
      b = i % _NBUF
      gather_wait(b)
      write_wait(b)
      scale(b)
      write_start(i, b)
    for b in range(_NBUF):
      write_wait(b)

  return emb_kernel


_emb_kernel = _make_sc_kernel()


@jax.jit
def kernel(tokens, table):
  wide = _tc_widen(table.T)
  flat = wide.reshape(2 * _VOCAB, _EMB)
  idx = tokens.reshape(_NW, _CHUNK, _N_CHUNKS).transpose(0, 2, 1)
  out = _emb_kernel(idx, flat)
  return out.reshape(_L, _B, _EMB).transpose(1, 0, 2)
